# baseline (speedup 1.0000x reference)
import functools
import jax
import jax.numpy as jnp
from jax import lax
from jax.experimental import pallas as pl
from jax.experimental.pallas import tpu as pltpu, tpu_sc as plsc

EMBED_DIM = 2048
NUM_EXPERTS = 16
N_TOKENS = 16384
BLK = 2048

NC, NS, L = 2, 16, 16
NW = NC * NS
CHUNK = N_TOKENS // NW


def _logits_body(x_ref, w_ref, b_ref, lt_ref):
    lt = jax.lax.dot_general(
        w_ref[...], x_ref[...], (((1,), (1,)), ((), ())),
        preferred_element_type=jnp.float32)
    lt_ref[...] = lt + b_ref[...]


def _logits_t(x, W, b):
    grid = (N_TOKENS // BLK,)
    return pl.pallas_call(
        _logits_body,
        grid=grid,
        in_specs=[
            pl.BlockSpec((BLK, EMBED_DIM), lambda i: (i, 0)),
            pl.BlockSpec((NUM_EXPERTS, EMBED_DIM), lambda i: (0, 0)),
            pl.BlockSpec((NUM_EXPERTS, 1), lambda i: (0, 0)),
        ],
        out_specs=pl.BlockSpec((NUM_EXPERTS, BLK), lambda i: (0, i)),
        out_shape=jax.ShapeDtypeStruct((NUM_EXPERTS, N_TOKENS), jnp.float32),
    )(x, W, b.reshape(NUM_EXPERTS, 1))


def _router(lt_hbm, gates_hbm, idx_hbm, lv, g1v, g2v, i1v, i2v):
    wid = lax.axis_index("s") * NC + lax.axis_index("c")
    base = wid * CHUNK
    pltpu.sync_copy(lt_hbm.at[:, pl.ds(base, CHUNK)], lv)

    def group(g, _):
        off = g * L
        m1 = lv[0, pl.ds(off, L)]
        i1 = jnp.zeros((L,), jnp.int32)
        m2 = jnp.full((L,), -jnp.inf, jnp.float32)
        i2 = jnp.zeros((L,), jnp.int32)
        for e in range(1, NUM_EXPERTS):
            v = lv[e, pl.ds(off, L)]
            ev = jnp.full((L,), e, jnp.int32)
            gt1 = v > m1
            gt2 = v > m2
            m2 = jnp.where(gt1, m1, jnp.where(gt2, v, m2))
            i2 = jnp.where(gt1, i1, jnp.where(gt2, ev, i2))
            m1 = jnp.where(gt1, v, m1)
            i1 = jnp.where(gt1, ev, i1)
        e2 = jnp.exp(m2 - m1)
        den = 1.0 + e2
        g1v[pl.ds(off, L)] = 1.0 / den
        g2v[pl.ds(off, L)] = e2 / den
        i1v[pl.ds(off, L)] = i1
        i2v[pl.ds(off, L)] = i2
        return 0

    lax.fori_loop(0, CHUNK // L, group, 0)
    pltpu.sync_copy(g1v, gates_hbm.at[0, pl.ds(base, CHUNK)])
    pltpu.sync_copy(g2v, gates_hbm.at[1, pl.ds(base, CHUNK)])
    pltpu.sync_copy(i1v, idx_hbm.at[0, pl.ds(base, CHUNK)])
    pltpu.sync_copy(i2v, idx_hbm.at[1, pl.ds(base, CHUNK)])


def _route(lt):
    mesh = plsc.VectorSubcoreMesh(core_axis_name="c", subcore_axis_name="s")
    f = functools.partial(
        pl.kernel, mesh=mesh,
        out_type=[
            jax.ShapeDtypeStruct((2, N_TOKENS), jnp.float32),
            jax.ShapeDtypeStruct((2, N_TOKENS), jnp.int32),
        ],
        scratch_types=[
            pltpu.VMEM((NUM_EXPERTS, CHUNK), jnp.float32),
            pltpu.VMEM((CHUNK,), jnp.float32),
            pltpu.VMEM((CHUNK,), jnp.float32),
            pltpu.VMEM((CHUNK,), jnp.int32),
            pltpu.VMEM((CHUNK,), jnp.int32),
        ],
    )(_router)
    return f(lt)


def kernel(x, W, b):
    lt = _logits_t(x, W, b)
    gates_t, idx_t = _route(lt)
    return (gates_t.T, idx_t.T)
